# Initial kernel scaffold; baseline (speedup 1.0000x reference)
import functools

import jax
import jax.numpy as jnp
from jax import lax
from jax.experimental import pallas as pl
from jax.experimental.pallas import tpu as pltpu
from jax.experimental.pallas import tpu_sc as plsc

N = 10000
E = 320000
DIM = 128
HEAD = 4
DH = DIM // HEAD
HP = 8

NC = 2
NS = 16
NW = NC * NS
EPW = E // NW
CH = 80
NCHUNK = EPW // CH
NGRP = CH // 16
RPT = N // NS


def _qkv_body(e_ref, w_ref, o_ref):
    o_ref[...] = jnp.dot(e_ref[...], w_ref[...],
                         preferred_element_type=jnp.float32)


def _qkv(embeds, wall):
    blk = 1000
    return pl.pallas_call(
        _qkv_body,
        grid=(N // blk,),
        in_specs=[
            pl.BlockSpec((blk, DIM), lambda i: (i, 0)),
            pl.BlockSpec((DIM, 3 * DIM), lambda i: (0, 0)),
        ],
        out_specs=pl.BlockSpec((blk, 3 * DIM), lambda i: (i, 0)),
        out_shape=jax.ShapeDtypeStruct((N, 3 * DIM), jnp.float32),
    )(embeds, wall)


def _edge_body(q_hbm, k_hbm, v_hbm, rows_hbm, cols_hbm, zres_hbm, znorm_hbm,
               exp_hbm, normp_hbm, resp_hbm,
               rows_v, cols_v, qb, kb, vb, sb, eb,
               res_sh, norm_sh, sem):
    cid = lax.axis_index("c")
    sid = lax.axis_index("s")
    wid = sid * NC + cid

    r0 = sid * RPT
    pltpu.sync_copy(zres_hbm.at[pl.ds(r0, RPT)], res_sh.at[pl.ds(r0, RPT)])
    pltpu.sync_copy(znorm_hbm.at[pl.ds(r0, RPT)], norm_sh.at[pl.ds(r0, RPT)])
    for i in range(CH * HP // 16):
        eb[pl.ds(i * 16, 16)] = jnp.zeros((16,), jnp.float32)
    plsc.subcore_barrier()

    lanes = lax.iota(jnp.int32, 16)

    def chunk(ci, _):
        base = wid * EPW + ci * CH
        pltpu.sync_copy(rows_hbm.at[pl.ds(base, CH)], rows_v)
        pltpu.sync_copy(cols_hbm.at[pl.ds(base, CH)], cols_v)
        pltpu.async_copy(q_hbm.at[rows_v], qb, sem).wait()
        pltpu.async_copy(k_hbm.at[cols_v], kb, sem).wait()
        pltpu.async_copy(v_hbm.at[cols_v], vb, sem).wait()

        for g in range(NGRP):
            eidx = lanes + (g * 16)
            for h in range(HEAD):
                def dot_d(d2, acc):
                    d = jnp.full((16,), h * DH + d2, jnp.int32)
                    qc = plsc.load_gather(qb, [eidx, d])
                    kc = plsc.load_gather(kb, [eidx, d])
                    return acc + qc * kc
                att = lax.fori_loop(0, DH, dot_d, jnp.zeros((16,), jnp.float32))
                att = jnp.exp(jnp.clip(att, -10.0, 10.0))
                plsc.store_scatter(eb, [eidx * HP + h], att)

                def scale_d(d2, t):
                    d = jnp.full((16,), h * DH + d2, jnp.int32)
                    vc = plsc.load_gather(vb, [eidx, d])
                    plsc.store_scatter(sb, [eidx, d], vc * att)
                    return t
                lax.fori_loop(0, DH, scale_d, 0)

        pltpu.sync_copy(sb, res_sh.at[rows_v], add=True)
        pltpu.sync_copy(eb, norm_sh.at[rows_v], add=True)
        pltpu.sync_copy(eb, exp_hbm.at[pl.ds(base, CH)])
        return 0

    lax.fori_loop(0, NCHUNK, chunk, 0)

    plsc.subcore_barrier()
    pltpu.sync_copy(res_sh.at[pl.ds(r0, RPT)],
                    resp_hbm.at[cid, pl.ds(r0, RPT)])
    pltpu.sync_copy(norm_sh.at[pl.ds(r0, RPT)],
                    normp_hbm.at[cid, pl.ds(r0, RPT)])


def _edge_pass(q, k, v, rows, cols):
    mesh = plsc.VectorSubcoreMesh(core_axis_name="c", subcore_axis_name="s")
    zres = jnp.zeros((N, DIM), jnp.float32)
    znorm = jnp.zeros((N, HP), jnp.float32)
    fn = functools.partial(
        pl.kernel,
        mesh=mesh,
        out_type=[
            jax.ShapeDtypeStruct((E, HP), jnp.float32),
            jax.ShapeDtypeStruct((NC, N, HP), jnp.float32),
            jax.ShapeDtypeStruct((NC, N, DIM), jnp.float32),
        ],
        scratch_types=[
            pltpu.VMEM((CH,), jnp.int32),
            pltpu.VMEM((CH,), jnp.int32),
            pltpu.VMEM((CH, DIM), jnp.float32),
            pltpu.VMEM((CH, DIM), jnp.float32),
            pltpu.VMEM((CH, DIM), jnp.float32),
            pltpu.VMEM((CH, DIM), jnp.float32),
            pltpu.VMEM((CH * HP,), jnp.float32),
            pltpu.VMEM_SHARED((N, DIM), jnp.float32),
            pltpu.VMEM_SHARED((N, HP), jnp.float32),
            pltpu.SemaphoreType.DMA,
        ],
    )(_edge_body)
    return fn(q, k, v, rows, cols, zres, znorm)


def _comb_body(resp_ref, normp_ref, res_ref, rn_ref):
    nsum = normp_ref[0] + normp_ref[1]
    rn = 1.0 / (nsum + 1e-8)
    rn_ref[...] = rn
    hsel = (lax.broadcasted_iota(jnp.int32, (HP, DIM), 0)
            == lax.broadcasted_iota(jnp.int32, (HP, DIM), 1) // DH)
    denom = jnp.dot(rn, hsel.astype(jnp.float32),
                    preferred_element_type=jnp.float32)
    res_ref[...] = (resp_ref[0] + resp_ref[1]) * denom


def _combine(resp, normp):
    blk = 1000
    return pl.pallas_call(
        _comb_body,
        grid=(N // blk,),
        in_specs=[
            pl.BlockSpec((NC, blk, DIM), lambda i: (0, i, 0)),
            pl.BlockSpec((NC, blk, HP), lambda i: (0, i, 0)),
        ],
        out_specs=[
            pl.BlockSpec((blk, DIM), lambda i: (i, 0)),
            pl.BlockSpec((blk, HP), lambda i: (i, 0)),
        ],
        out_shape=[
            jax.ShapeDtypeStruct((N, DIM), jnp.float32),
            jax.ShapeDtypeStruct((N, HP), jnp.float32),
        ],
    )(resp, normp)


def _att_body(exp_hbm, rows_hbm, rn_hbm, att_hbm, rn_v, rows_v, exp_v, out_v):
    cid = lax.axis_index("c")
    sid = lax.axis_index("s")
    wid = sid * NC + cid
    pltpu.sync_copy(rn_hbm, rn_v)
    lanes = lax.iota(jnp.int32, 16)

    def chunk(ci, _):
        base = wid * EPW + ci * CH
        pltpu.sync_copy(rows_hbm.at[pl.ds(base, CH)], rows_v)
        pltpu.sync_copy(exp_hbm.at[pl.ds(base * HP, CH * HP)], exp_v)
        for g in range(NGRP):
            eidx = lanes + (g * 16)
            ri = plsc.load_gather(rows_v, [eidx])
            for h in range(HEAD):
                ev = plsc.load_gather(exp_v, [eidx * HP + h])
                rv = plsc.load_gather(rn_v, [ri * HP + h])
                plsc.store_scatter(out_v, [eidx * HEAD + h], ev * rv)
        pltpu.sync_copy(out_v, att_hbm.at[pl.ds(base, CH)])
        return 0

    lax.fori_loop(0, NCHUNK, chunk, 0)


def _att_norm(exp_flat, rows, rn_flat):
    mesh = plsc.VectorSubcoreMesh(core_axis_name="c", subcore_axis_name="s")
    fn = functools.partial(
        pl.kernel,
        mesh=mesh,
        out_type=jax.ShapeDtypeStruct((E, HEAD), jnp.float32),
        scratch_types=[
            pltpu.VMEM((N * HP,), jnp.float32),
            pltpu.VMEM((CH,), jnp.int32),
            pltpu.VMEM((CH * HP,), jnp.float32),
            pltpu.VMEM((CH, HEAD), jnp.float32),
        ],
    )(_att_body)
    return fn(exp_flat, rows, rn_flat)


def kernel(embeds, edge_index, qW, kW, vW):
    rows = edge_index[0]
    cols = edge_index[1]
    wall = jnp.concatenate([qW, kW, vW], axis=1)
    qkv = _qkv(embeds, wall)
    q = qkv[:, :DIM]
    k = qkv[:, DIM:2 * DIM]
    v = qkv[:, 2 * DIM:]
    expatt, normp, resp = _edge_pass(q, k, v, rows, cols)
    res, rn = _combine(resp, normp)
    att = _att_norm(expatt.reshape(E * HP), rows, rn.reshape(N * HP))
    return res, att


# speedup vs baseline: 1.2513x; 1.2513x over previous
import functools

import jax
import jax.numpy as jnp
from jax import lax
from jax.experimental import pallas as pl
from jax.experimental.pallas import tpu as pltpu
from jax.experimental.pallas import tpu_sc as plsc

N = 10000
E = 320000
DIM = 128
HEAD = 4
DH = DIM // HEAD
HP = 8

NC = 2
NS = 16
NW = NC * NS
EPW = E // NW
CH = 80
NCHUNK = EPW // CH
NGRP = CH // 16
RPT = 624
RTAIL = N - NS * RPT

_SC_PARAMS = pltpu.CompilerParams(needs_layout_passes=False)


def _qkv_body(e_ref, w_ref, o_ref):
    o_ref[...] = jnp.dot(e_ref[...], w_ref[...],
                         preferred_element_type=jnp.float32)


def _qkv(embeds, wall):
    blk = 1000
    return pl.pallas_call(
        _qkv_body,
        grid=(N // blk,),
        in_specs=[
            pl.BlockSpec((blk, DIM), lambda i: (i, 0)),
            pl.BlockSpec((DIM, 3 * DIM), lambda i: (0, 0)),
        ],
        out_specs=pl.BlockSpec((blk, 3 * DIM), lambda i: (i, 0)),
        out_shape=jax.ShapeDtypeStruct((N, 3 * DIM), jnp.float32),
    )(embeds, wall)


def _edge_body(q_hbm, k_hbm, v_hbm, rows_hbm, cols_hbm, zres_hbm, znorm_hbm,
               exp_hbm, normp_hbm, resp_hbm,
               rows_v, cols_v, qb, kb, vb, eb, nidx,
               res_sh, norm_sh, sem):
    cid = lax.axis_index("c")
    sid = lax.axis_index("s")
    wid = sid * NC + cid

    r0 = sid * RPT
    pltpu.sync_copy(zres_hbm.at[pl.ds(r0, RPT)], res_sh.at[pl.ds(r0, RPT)])
    pltpu.sync_copy(znorm_hbm.at[pl.ds(r0 * HP, RPT * HP)],
                    norm_sh.at[pl.ds(r0 * HP, RPT * HP)])

    @pl.when(sid == NS - 1)
    def _():
        t0 = NS * RPT
        pltpu.sync_copy(zres_hbm.at[pl.ds(t0, RTAIL)],
                        res_sh.at[pl.ds(t0, RTAIL)])
        pltpu.sync_copy(znorm_hbm.at[pl.ds(t0 * HP, RTAIL * HP)],
                        norm_sh.at[pl.ds(t0 * HP, RTAIL * HP)])

    lanes = lax.iota(jnp.int32, 16)
    zero16 = jnp.zeros((16,), jnp.float32)
    for g in range(NGRP):
        gv = jnp.full((16,), g, jnp.int32)
        for h in range(HEAD, HP):
            plsc.store_scatter(eb, [gv, lanes * HP + h], zero16)
    plsc.subcore_barrier()

    def chunk(ci, _):
        base = wid * EPW + ci * CH
        pltpu.sync_copy(rows_hbm.at[pl.ds(base, CH)], rows_v)
        pltpu.sync_copy(cols_hbm.at[pl.ds(base, CH)], cols_v)
        pltpu.async_copy(q_hbm.at[rows_v], qb, sem).wait()
        pltpu.async_copy(k_hbm.at[cols_v], kb, sem).wait()
        pltpu.async_copy(v_hbm.at[cols_v], vb, sem).wait()

        for g in range(NGRP):
            eidx = lanes + (g * 16)
            gv = jnp.full((16,), g, jnp.int32)
            for h in range(HEAD):
                def dot_d(d2, acc):
                    d = jnp.full((16,), h * DH + d2, jnp.int32)
                    qc = plsc.load_gather(qb, [eidx, d])
                    kc = plsc.load_gather(kb, [eidx, d])
                    return acc + qc * kc
                att = lax.fori_loop(0, DH, dot_d, jnp.zeros((16,), jnp.float32))
                att = jnp.exp(jnp.clip(att, -10.0, 10.0))
                plsc.store_scatter(eb, [gv, lanes * HP + h], att)

                def scale_d(d2, t):
                    d = jnp.full((16,), h * DH + d2, jnp.int32)
                    vc = plsc.load_gather(vb, [eidx, d])
                    plsc.store_scatter(vb, [eidx, d], vc * att)
                    return t
                lax.fori_loop(0, DH, scale_d, 0)

            for i in range(HP):
                glob = lanes + i * 16
                evec = (glob >> 3) + g * 16
                hvec = glob & (HP - 1)
                rv = plsc.load_gather(rows_v, [evec])
                nidx[g, pl.ds(i * 16, 16)] = rv * HP + hvec

        pltpu.sync_copy(vb, res_sh.at[rows_v], add=True)
        for g in range(NGRP):
            pltpu.sync_copy(eb.at[g], norm_sh.at[nidx.at[g]], add=True)
            pltpu.sync_copy(eb.at[g],
                            exp_hbm.at[pl.ds(base * HP + g * 128, 128)])
        return 0

    lax.fori_loop(0, NCHUNK, chunk, 0)

    plsc.subcore_barrier()
    pltpu.sync_copy(res_sh.at[pl.ds(r0, RPT)],
                    resp_hbm.at[cid, pl.ds(r0, RPT)])
    pltpu.sync_copy(norm_sh.at[pl.ds(r0 * HP, RPT * HP)],
                    normp_hbm.at[pl.ds(cid * N * HP + r0 * HP, RPT * HP)])

    @pl.when(sid == NS - 1)
    def _():
        t0 = NS * RPT
        pltpu.sync_copy(res_sh.at[pl.ds(t0, RTAIL)],
                        resp_hbm.at[cid, pl.ds(t0, RTAIL)])
        pltpu.sync_copy(norm_sh.at[pl.ds(t0 * HP, RTAIL * HP)],
                        normp_hbm.at[pl.ds(cid * N * HP + t0 * HP,
                                           RTAIL * HP)])


def _edge_pass(q, k, v, rows, cols):
    mesh = plsc.VectorSubcoreMesh(core_axis_name="c", subcore_axis_name="s")
    zres = jnp.zeros((N, DIM), jnp.float32)
    znorm = jnp.zeros((N * HP,), jnp.float32)
    fn = functools.partial(
        pl.kernel,
        mesh=mesh,
        compiler_params=_SC_PARAMS,
        out_type=[
            jax.ShapeDtypeStruct((E * HP,), jnp.float32),
            jax.ShapeDtypeStruct((NC * N * HP,), jnp.float32),
            jax.ShapeDtypeStruct((NC, N, DIM), jnp.float32),
        ],
        scratch_types=[
            pltpu.VMEM((CH,), jnp.int32),
            pltpu.VMEM((CH,), jnp.int32),
            pltpu.VMEM((CH, DIM), jnp.float32),
            pltpu.VMEM((CH, DIM), jnp.float32),
            pltpu.VMEM((CH, DIM), jnp.float32),
            pltpu.VMEM((NGRP, 128), jnp.float32),
            pltpu.VMEM((NGRP, 128), jnp.int32),
            pltpu.VMEM_SHARED((N, DIM), jnp.float32),
            pltpu.VMEM_SHARED((N * HP,), jnp.float32),
            pltpu.SemaphoreType.DMA,
        ],
    )(_edge_body)
    return fn(q, k, v, rows, cols, zres, znorm)


def _comb_body(resp_ref, normp_ref, res_ref, rn_ref):
    nsum = normp_ref[0] + normp_ref[1]
    rn = 1.0 / (nsum + 1e-8)
    rn_ref[...] = rn
    hsel = (lax.broadcasted_iota(jnp.int32, (HP, DIM), 0)
            == lax.broadcasted_iota(jnp.int32, (HP, DIM), 1) // DH)
    denom = jnp.dot(rn, hsel.astype(jnp.float32),
                    preferred_element_type=jnp.float32)
    res_ref[...] = (resp_ref[0] + resp_ref[1]) * denom


def _combine(resp, normp):
    blk = 1000
    return pl.pallas_call(
        _comb_body,
        grid=(N // blk,),
        in_specs=[
            pl.BlockSpec((NC, blk, DIM), lambda i: (0, i, 0)),
            pl.BlockSpec((NC, blk, HP), lambda i: (0, i, 0)),
        ],
        out_specs=[
            pl.BlockSpec((blk, DIM), lambda i: (i, 0)),
            pl.BlockSpec((blk, HP), lambda i: (i, 0)),
        ],
        out_shape=[
            jax.ShapeDtypeStruct((N, DIM), jnp.float32),
            jax.ShapeDtypeStruct((N, HP), jnp.float32),
        ],
    )(resp, normp)


def _att_body(exp_hbm, rows_hbm, rn_hbm, att_hbm, rn_v, rows_v, exp_v, out_v):
    cid = lax.axis_index("c")
    sid = lax.axis_index("s")
    wid = sid * NC + cid
    pltpu.sync_copy(rn_hbm, rn_v)
    lanes = lax.iota(jnp.int32, 16)

    def chunk(ci, _):
        base = wid * EPW + ci * CH
        pltpu.sync_copy(rows_hbm.at[pl.ds(base, CH)], rows_v)
        pltpu.sync_copy(exp_hbm.at[pl.ds(base * HP, CH * HP)], exp_v)
        for g in range(NGRP):
            eidx = lanes + (g * 16)
            ri = plsc.load_gather(rows_v, [eidx])
            for h in range(HEAD):
                ev = plsc.load_gather(exp_v, [eidx * HP + h])
                rv = plsc.load_gather(rn_v, [ri * HP + h])
                plsc.store_scatter(out_v, [eidx * HEAD + h], ev * rv)
        pltpu.sync_copy(out_v, att_hbm.at[pl.ds(base * HEAD, CH * HEAD)])
        return 0

    lax.fori_loop(0, NCHUNK, chunk, 0)


def _att_norm(exp_flat, rows, rn_flat):
    mesh = plsc.VectorSubcoreMesh(core_axis_name="c", subcore_axis_name="s")
    fn = functools.partial(
        pl.kernel,
        mesh=mesh,
        compiler_params=_SC_PARAMS,
        out_type=jax.ShapeDtypeStruct((E * HEAD,), jnp.float32),
        scratch_types=[
            pltpu.VMEM((N * HP,), jnp.float32),
            pltpu.VMEM((CH,), jnp.int32),
            pltpu.VMEM((CH * HP,), jnp.float32),
            pltpu.VMEM((CH * HEAD,), jnp.float32),
        ],
    )(_att_body)
    return fn(exp_flat, rows, rn_flat)


def kernel(embeds, edge_index, qW, kW, vW):
    rows = edge_index[0]
    cols = edge_index[1]
    wall = jnp.concatenate([qW, kW, vW], axis=1)
    qkv = _qkv(embeds, wall)
    q = qkv[:, :DIM]
    k = qkv[:, DIM:2 * DIM]
    v = qkv[:, 2 * DIM:]
    exp_flat, normp, resp = _edge_pass(q, k, v, rows, cols)
    res, rn = _combine(resp, normp.reshape(NC, N, HP))
    att = _att_norm(exp_flat, rows, rn.reshape(N * HP))
    return res, att.reshape(E, HEAD)


# speedup vs baseline: 1.4059x; 1.1236x over previous
import functools

import jax
import jax.numpy as jnp
from jax import lax
from jax.experimental import pallas as pl
from jax.experimental.pallas import tpu as pltpu
from jax.experimental.pallas import tpu_sc as plsc

N = 10000
E = 320000
DIM = 128
HEAD = 4
DH = DIM // HEAD
N4 = N * HEAD

NC = 2
NS = 16
NW = NC * NS
EPW = E // NW
CH = 192
NG = CH // 16
SG = 48
NSUB = CH // SG
NCHUNK = EPW // CH
TB = NCHUNK * CH
CT = EPW - TB
RPT = 624
RTAIL = N - NS * RPT

_SC_PARAMS = pltpu.CompilerParams(needs_layout_passes=False)


def _qkv_body(e_ref, w_ref, o_ref):
    o_ref[...] = jnp.dot(e_ref[...], w_ref[...],
                         preferred_element_type=jnp.float32)


def _qkv(embeds, wall):
    blk = 1000
    return pl.pallas_call(
        _qkv_body,
        grid=(N // blk,),
        in_specs=[
            pl.BlockSpec((blk, DIM), lambda i: (i, 0)),
            pl.BlockSpec((DIM, 3 * DIM), lambda i: (0, 0)),
        ],
        out_specs=pl.BlockSpec((blk, 3 * DIM), lambda i: (i, 0)),
        out_shape=jax.ShapeDtypeStruct((N, 3 * DIM), jnp.float32),
    )(embeds, wall)


def _score_body(q_hbm, k_hbm, rows_hbm, cols_hbm,
                exp_hbm, normp_hbm,
                rows_all, cols_all, qb, kb, qt, kt, eb, ntab, gsem, osem):
    cid = lax.axis_index("c")
    sid = lax.axis_index("s")
    wid = sid * NC + cid
    wbase = wid * EPW
    lanes = lax.iota(jnp.int32, 16)
    zero16 = jnp.zeros((16,), jnp.float32)

    def ztab(i, _):
        ntab[pl.ds(i * 16, 16)] = zero16
        return 0
    lax.fori_loop(0, N4 // 16, ztab, 0)

    d0 = pltpu.async_copy(rows_hbm.at[pl.ds(wbase, EPW)], rows_all, gsem)
    d1 = pltpu.async_copy(cols_hbm.at[pl.ds(wbase, EPW)], cols_all, gsem)
    d0.wait()
    d1.wait()

    def heads(buf_q, buf_k, loc0, g):
        eloc = lanes + g * 16
        eabs = eloc + loc0
        for h in range(HEAD):
            def dot_d(d2, acc):
                dd = h * DH + d2 * 4
                for t in range(4):
                    d = jnp.full((16,), dd + t, jnp.int32)
                    qc = plsc.load_gather(buf_q, [eloc, d])
                    kc = plsc.load_gather(buf_k, [eloc, d])
                    acc = acc + qc * kc
                return acc
            att = lax.fori_loop(0, DH // 4, dot_d,
                                jnp.zeros((16,), jnp.float32))
            att = jnp.exp(jnp.clip(att, -10.0, 10.0))
            eb[pl.ds(g * 64 + h * 16, 16)] = att
            rv = plsc.load_gather(rows_all, [eabs])
            plsc.addupdate_scatter(ntab, [rv * HEAD + h], att)

    def chunk(ci, _):
        loc = ci * CH
        descs = []
        for j in range(NSUB):
            descs.append(pltpu.async_copy(
                q_hbm.at[rows_all.at[pl.ds(loc + j * SG, SG)]],
                qb.at[pl.ds(j * SG, SG)], gsem))
            descs.append(pltpu.async_copy(
                k_hbm.at[cols_all.at[pl.ds(loc + j * SG, SG)]],
                kb.at[pl.ds(j * SG, SG)], gsem))
        for d in descs:
            d.wait()

        def group(g, __):
            heads(qb, kb, loc, g)
            return 0
        lax.fori_loop(0, NG, group, 0)

        od = pltpu.async_copy(
            eb, exp_hbm.at[pl.ds((wbase + ci * CH) * HEAD, CH * HEAD)], osem)
        od.wait()
        return 0

    lax.fori_loop(0, NCHUNK, chunk, 0)

    dt0 = pltpu.async_copy(q_hbm.at[rows_all.at[pl.ds(TB, CT)]], qt, gsem)
    dt1 = pltpu.async_copy(k_hbm.at[cols_all.at[pl.ds(TB, CT)]], kt, gsem)
    dt0.wait()
    dt1.wait()
    heads(qt, kt, TB, 0)
    pltpu.async_copy(eb.at[pl.ds(0, CT * HEAD)],
                     exp_hbm.at[pl.ds((wbase + TB) * HEAD, CT * HEAD)],
                     osem).wait()

    pltpu.sync_copy(ntab, normp_hbm.at[pl.ds(wid * N4, N4)])


def _score_pass(q, k, rows, cols):
    mesh = plsc.VectorSubcoreMesh(core_axis_name="c", subcore_axis_name="s")
    fn = functools.partial(
        pl.kernel,
        mesh=mesh,
        compiler_params=_SC_PARAMS,
        out_type=[
            jax.ShapeDtypeStruct((E * HEAD,), jnp.float32),
            jax.ShapeDtypeStruct((NW * N4,), jnp.float32),
        ],
        scratch_types=[
            pltpu.VMEM((EPW,), jnp.int32),
            pltpu.VMEM((EPW,), jnp.int32),
            pltpu.VMEM((CH, DIM), jnp.float32),
            pltpu.VMEM((CH, DIM), jnp.float32),
            pltpu.VMEM((CT, DIM), jnp.float32),
            pltpu.VMEM((CT, DIM), jnp.float32),
            pltpu.VMEM((CH * HEAD,), jnp.float32),
            pltpu.VMEM((N4,), jnp.float32),
            pltpu.SemaphoreType.DMA,
            pltpu.SemaphoreType.DMA,
        ],
    )(_score_body)
    return fn(q, k, rows, cols)


def _agg_body(v_hbm, rows_hbm, cols_hbm, exp_hbm, zres_hbm,
              resp_hbm,
              cols_all, rows_c, rows_t, vb, vt, eb,
              rs0, rs1, rs2, rs3,
              res_sh, gsem, osem):
    cid = lax.axis_index("c")
    sid = lax.axis_index("s")
    wid = sid * NC + cid
    wbase = wid * EPW
    lanes = lax.iota(jnp.int32, 16)
    rsub = [rs0, rs1, rs2, rs3]

    t0 = NS * RPT
    p0 = sid * RPT
    pltpu.sync_copy(zres_hbm.at[pl.ds(p0, RPT)], res_sh.at[pl.ds(p0, RPT)])

    @pl.when(sid == NS - 1)
    def _():
        pltpu.sync_copy(zres_hbm.at[pl.ds(t0, RTAIL)],
                        res_sh.at[pl.ds(t0, RTAIL)])

    d0 = pltpu.async_copy(cols_hbm.at[pl.ds(wbase, EPW)], cols_all, gsem)
    d0.wait()
    plsc.subcore_barrier()

    def scale(buf_v, buf_e, g):
        eidx = lanes + g * 16
        for h in range(HEAD):
            att = plsc.load_gather(buf_e, [g * 64 + h * 16 + lanes])

            def scale_d(d2, tt):
                dd = h * DH + d2 * 4
                for t in range(4):
                    d = jnp.full((16,), dd + t, jnp.int32)
                    vc = plsc.load_gather(buf_v, [eidx, d])
                    plsc.store_scatter(buf_v, [eidx, d], vc * att)
                return tt
            lax.fori_loop(0, DH // 4, scale_d, 0)

    def chunk(ci, _):
        base = wbase + ci * CH
        descs = [
            pltpu.async_copy(rows_hbm.at[pl.ds(base, CH)], rows_c, gsem),
            pltpu.async_copy(exp_hbm.at[pl.ds(base * HEAD, CH * HEAD)],
                             eb, gsem),
        ]
        for j in range(NSUB):
            descs.append(pltpu.async_copy(
                v_hbm.at[cols_all.at[pl.ds(ci * CH + j * SG, SG)]],
                vb.at[pl.ds(j * SG, SG)], gsem))
        for d in descs:
            d.wait()

        def group(g, __):
            scale(vb, eb, g)
            return 0
        lax.fori_loop(0, NG, group, 0)

        for j in range(NG):
            rv = plsc.load_gather(rows_c, [lanes + j * 16])
            rsub[j // (SG // 16)][pl.ds((j % (SG // 16)) * 16, 16)] = rv

        odescs = []
        for j in range(NSUB):
            odescs.append(pltpu.async_copy(
                vb.at[pl.ds(j * SG, SG)], res_sh.at[rsub[j]], osem,
                add=True))
        for d in odescs:
            d.wait()
        return 0

    lax.fori_loop(0, NCHUNK, chunk, 0)

    tb = wbase + TB
    dts = [
        pltpu.async_copy(rows_hbm.at[pl.ds(tb, CT)], rows_t, gsem),
        pltpu.async_copy(exp_hbm.at[pl.ds(tb * HEAD, CT * HEAD)],
                         eb.at[pl.ds(0, CT * HEAD)], gsem),
        pltpu.async_copy(v_hbm.at[cols_all.at[pl.ds(TB, CT)]], vt, gsem),
    ]
    for d in dts:
        d.wait()
    scale(vt, eb, 0)
    pltpu.async_copy(vt, res_sh.at[rows_t], osem, add=True).wait()

    plsc.subcore_barrier()
    pltpu.sync_copy(res_sh.at[pl.ds(p0, RPT)],
                    resp_hbm.at[cid, pl.ds(p0, RPT)])

    @pl.when(sid == NS - 1)
    def _():
        pltpu.sync_copy(res_sh.at[pl.ds(t0, RTAIL)],
                        resp_hbm.at[cid, pl.ds(t0, RTAIL)])


def _agg_pass(v, rows, cols, exp_flat):
    mesh = plsc.VectorSubcoreMesh(core_axis_name="c", subcore_axis_name="s")
    zres = jnp.zeros((N, DIM), jnp.float32)
    fn = functools.partial(
        pl.kernel,
        mesh=mesh,
        compiler_params=_SC_PARAMS,
        out_type=jax.ShapeDtypeStruct((NC, N, DIM), jnp.float32),
        scratch_types=[
            pltpu.VMEM((EPW,), jnp.int32),
            pltpu.VMEM((CH,), jnp.int32),
            pltpu.VMEM((CT,), jnp.int32),
            pltpu.VMEM((CH, DIM), jnp.float32),
            pltpu.VMEM((CT, DIM), jnp.float32),
            pltpu.VMEM((CH * HEAD,), jnp.float32),
            pltpu.VMEM((SG,), jnp.int32),
            pltpu.VMEM((SG,), jnp.int32),
            pltpu.VMEM((SG,), jnp.int32),
            pltpu.VMEM((SG,), jnp.int32),
            pltpu.VMEM_SHARED((N, DIM), jnp.float32),
            pltpu.SemaphoreType.DMA,
            pltpu.SemaphoreType.DMA,
        ],
    )(_agg_body)
    return fn(v, rows, cols, exp_flat, zres)


def _comb_body(resp_ref, normp_ref, res_ref, rn_ref):
    nsum = jnp.sum(normp_ref[...], axis=0)
    rn = 1.0 / (nsum + 1e-8)
    rn_ref[...] = rn
    hsel = (lax.broadcasted_iota(jnp.int32, (HEAD, DIM), 0)
            == lax.broadcasted_iota(jnp.int32, (HEAD, DIM), 1) // DH)
    denom = jnp.dot(rn, hsel.astype(jnp.float32),
                    preferred_element_type=jnp.float32)
    res_ref[...] = (resp_ref[0] + resp_ref[1]) * denom


def _combine(resp, normp):
    blk = 1000
    return pl.pallas_call(
        _comb_body,
        grid=(N // blk,),
        in_specs=[
            pl.BlockSpec((NC, blk, DIM), lambda i: (0, i, 0)),
            pl.BlockSpec((NW, blk, HEAD), lambda i: (0, i, 0)),
        ],
        out_specs=[
            pl.BlockSpec((blk, DIM), lambda i: (i, 0)),
            pl.BlockSpec((blk, HEAD), lambda i: (i, 0)),
        ],
        out_shape=[
            jax.ShapeDtypeStruct((N, DIM), jnp.float32),
            jax.ShapeDtypeStruct((N, HEAD), jnp.float32),
        ],
    )(resp, normp)


CH4 = 2000
NCHUNK4 = EPW // CH4


def _att_body(exp_hbm, rows_hbm, rn_hbm, att_hbm, rn_v, rows_v, exp_v, out_v,
              gsem):
    cid = lax.axis_index("c")
    sid = lax.axis_index("s")
    wid = sid * NC + cid
    pltpu.sync_copy(rn_hbm, rn_v)
    lanes = lax.iota(jnp.int32, 16)

    def chunk(ci, _):
        base = wid * EPW + ci * CH4
        d0 = pltpu.async_copy(rows_hbm.at[pl.ds(base, CH4)], rows_v, gsem)
        d1 = pltpu.async_copy(exp_hbm.at[pl.ds(base * HEAD, CH4 * HEAD)],
                              exp_v, gsem)
        d0.wait()
        d1.wait()

        def group(g, __):
            eidx = lanes + g * 16
            ri = plsc.load_gather(rows_v, [eidx])
            for h in range(HEAD):
                ev = plsc.load_gather(exp_v, [g * 64 + h * 16 + lanes])
                rv = plsc.load_gather(rn_v, [ri * HEAD + h])
                plsc.store_scatter(out_v, [eidx * HEAD + h], ev * rv)
            return 0

        lax.fori_loop(0, CH4 // 16, group, 0)
        pltpu.async_copy(out_v,
                         att_hbm.at[pl.ds(base * HEAD, CH4 * HEAD)],
                         gsem).wait()
        return 0

    lax.fori_loop(0, NCHUNK4, chunk, 0)


def _att_norm(exp_flat, rows, rn_flat):
    mesh = plsc.VectorSubcoreMesh(core_axis_name="c", subcore_axis_name="s")
    fn = functools.partial(
        pl.kernel,
        mesh=mesh,
        compiler_params=_SC_PARAMS,
        out_type=jax.ShapeDtypeStruct((E * HEAD,), jnp.float32),
        scratch_types=[
            pltpu.VMEM((N4,), jnp.float32),
            pltpu.VMEM((CH4,), jnp.int32),
            pltpu.VMEM((CH4 * HEAD,), jnp.float32),
            pltpu.VMEM((CH4 * HEAD,), jnp.float32),
            pltpu.SemaphoreType.DMA,
        ],
    )(_att_body)
    return fn(exp_flat, rows, rn_flat)


def kernel(embeds, edge_index, qW, kW, vW):
    rows = edge_index[0]
    cols = edge_index[1]
    wall = jnp.concatenate([qW, kW, vW], axis=1)
    qkv = _qkv(embeds, wall)
    q = qkv[:, :DIM]
    k = qkv[:, DIM:2 * DIM]
    v = qkv[:, 2 * DIM:]
    exp_flat, normp = _score_pass(q, k, rows, cols)
    resp = _agg_pass(v, rows, cols, exp_flat)
    res, rn = _combine(resp, normp.reshape(NW, N, HEAD))
    att = _att_norm(exp_flat, rows, rn.reshape(N4))
    return res, att.reshape(E, HEAD)


# speedup vs baseline: 3.8678x; 2.7511x over previous
import functools

import jax
import jax.numpy as jnp
from jax import lax
from jax.experimental import pallas as pl
from jax.experimental.pallas import tpu as pltpu
from jax.experimental.pallas import tpu_sc as plsc

N = 10000
E = 320000
DIM = 128
HEAD = 4
DH = DIM // HEAD
N4 = N * HEAD

NC = 2
NS = 16
NW = NC * NS
EPW = E // NW
CH = 192
NG = CH // 16
SG = 48
NSUB = CH // SG
NCHUNK = EPW // CH
TB = NCHUNK * CH
CT = EPW - TB
RPT = 624
RTAIL = N - NS * RPT

_SC_PARAMS = pltpu.CompilerParams(needs_layout_passes=False)


def _qkv_body(e_ref, w_ref, o_ref):
    o_ref[...] = jnp.dot(e_ref[...], w_ref[...],
                         preferred_element_type=jnp.float32)


def _qkv(embeds, wall):
    blk = 1000
    return pl.pallas_call(
        _qkv_body,
        grid=(N // blk,),
        in_specs=[
            pl.BlockSpec((blk, DIM), lambda i: (i, 0)),
            pl.BlockSpec((DIM, 3 * DIM), lambda i: (0, 0)),
        ],
        out_specs=pl.BlockSpec((blk, 3 * DIM), lambda i: (i, 0)),
        out_shape=jax.ShapeDtypeStruct((N, 3 * DIM), jnp.float32),
    )(embeds, wall)


def _score_body(q_hbm, k_hbm, rows_hbm, cols_hbm,
                exp_hbm, normp_hbm,
                rows_all, cols_all, qb, kb, qt, kt, eb, ntab, gsem, osem):
    cid = lax.axis_index("c")
    sid = lax.axis_index("s")
    wid = sid * NC + cid
    wbase = wid * EPW
    lanes = lax.iota(jnp.int32, 16)
    zero16 = jnp.zeros((16,), jnp.float32)

    def ztab(i, _):
        ntab[pl.ds(i * 16, 16)] = zero16
        return 0
    lax.fori_loop(0, N4 // 16, ztab, 0)

    d0 = pltpu.async_copy(rows_hbm.at[pl.ds(wbase, EPW)], rows_all, gsem)
    d1 = pltpu.async_copy(cols_hbm.at[pl.ds(wbase, EPW)], cols_all, gsem)
    d0.wait()
    d1.wait()

    def heads(buf_q, buf_k, loc0, g):
        eloc = lanes + g * 16
        eabs = eloc + loc0
        for h in range(HEAD):
            def dot_d(d2, acc):
                dd = d2 * 4
                for t in range(4):
                    d = (h * DH) + ((lanes + dd + t) & (DH - 1))
                    qc = plsc.load_gather(buf_q, [eloc, d])
                    kc = plsc.load_gather(buf_k, [eloc, d])
                    acc = acc + qc * kc
                return acc
            att = lax.fori_loop(0, DH // 4, dot_d,
                                jnp.zeros((16,), jnp.float32))
            att = jnp.exp(jnp.clip(att, -10.0, 10.0))
            eb[pl.ds(g * 64 + h * 16, 16)] = att
            rv = plsc.load_gather(rows_all, [eabs])
            plsc.addupdate_scatter(ntab, [rv * HEAD + h], att)

    def chunk(ci, _):
        loc = ci * CH
        descs = []
        for j in range(NSUB):
            descs.append(pltpu.async_copy(
                q_hbm.at[rows_all.at[pl.ds(loc + j * SG, SG)]],
                qb.at[pl.ds(j * SG, SG)], gsem))
            descs.append(pltpu.async_copy(
                k_hbm.at[cols_all.at[pl.ds(loc + j * SG, SG)]],
                kb.at[pl.ds(j * SG, SG)], gsem))
        for d in descs:
            d.wait()

        def group(g, __):
            heads(qb, kb, loc, g)
            return 0
        lax.fori_loop(0, NG, group, 0)

        od = pltpu.async_copy(
            eb, exp_hbm.at[pl.ds((wbase + ci * CH) * HEAD, CH * HEAD)], osem)
        od.wait()
        return 0

    lax.fori_loop(0, NCHUNK, chunk, 0)

    dt0 = pltpu.async_copy(q_hbm.at[rows_all.at[pl.ds(TB, CT)]], qt, gsem)
    dt1 = pltpu.async_copy(k_hbm.at[cols_all.at[pl.ds(TB, CT)]], kt, gsem)
    dt0.wait()
    dt1.wait()
    heads(qt, kt, TB, 0)
    pltpu.async_copy(eb.at[pl.ds(0, CT * HEAD)],
                     exp_hbm.at[pl.ds((wbase + TB) * HEAD, CT * HEAD)],
                     osem).wait()

    pltpu.sync_copy(ntab, normp_hbm.at[pl.ds(wid * N4, N4)])


def _score_pass(q, k, rows, cols):
    mesh = plsc.VectorSubcoreMesh(core_axis_name="c", subcore_axis_name="s")
    fn = functools.partial(
        pl.kernel,
        mesh=mesh,
        compiler_params=_SC_PARAMS,
        out_type=[
            jax.ShapeDtypeStruct((E * HEAD,), jnp.float32),
            jax.ShapeDtypeStruct((NW * N4,), jnp.float32),
        ],
        scratch_types=[
            pltpu.VMEM((EPW,), jnp.int32),
            pltpu.VMEM((EPW,), jnp.int32),
            pltpu.VMEM((CH, DIM), jnp.float32),
            pltpu.VMEM((CH, DIM), jnp.float32),
            pltpu.VMEM((CT, DIM), jnp.float32),
            pltpu.VMEM((CT, DIM), jnp.float32),
            pltpu.VMEM((CH * HEAD,), jnp.float32),
            pltpu.VMEM((N4,), jnp.float32),
            pltpu.SemaphoreType.DMA,
            pltpu.SemaphoreType.DMA,
        ],
    )(_score_body)
    return fn(q, k, rows, cols)


def _agg_body(v_hbm, rows_hbm, cols_hbm, exp_hbm, zres_hbm,
              resp_hbm,
              cols_all, rows_c, rows_t, vb, vt, eb,
              rs0, rs1, rs2, rs3,
              res_sh, gsem, osem):
    cid = lax.axis_index("c")
    sid = lax.axis_index("s")
    wid = sid * NC + cid
    wbase = wid * EPW
    lanes = lax.iota(jnp.int32, 16)
    rsub = [rs0, rs1, rs2, rs3]

    t0 = NS * RPT
    p0 = sid * RPT
    pltpu.sync_copy(zres_hbm.at[pl.ds(p0, RPT)], res_sh.at[pl.ds(p0, RPT)])

    @pl.when(sid == NS - 1)
    def _():
        pltpu.sync_copy(zres_hbm.at[pl.ds(t0, RTAIL)],
                        res_sh.at[pl.ds(t0, RTAIL)])

    d0 = pltpu.async_copy(cols_hbm.at[pl.ds(wbase, EPW)], cols_all, gsem)
    d0.wait()
    plsc.subcore_barrier()

    def scale(buf_v, buf_e, g):
        eidx = lanes + g * 16
        for h in range(HEAD):
            att = plsc.load_gather(buf_e, [g * 64 + h * 16 + lanes])

            def scale_d(d2, tt):
                dd = d2 * 4
                for t in range(4):
                    d = (h * DH) + ((lanes + dd + t) & (DH - 1))
                    vc = plsc.load_gather(buf_v, [eidx, d])
                    plsc.store_scatter(buf_v, [eidx, d], vc * att)
                return tt
            lax.fori_loop(0, DH // 4, scale_d, 0)

    def chunk(ci, _):
        base = wbase + ci * CH
        descs = [
            pltpu.async_copy(rows_hbm.at[pl.ds(base, CH)], rows_c, gsem),
            pltpu.async_copy(exp_hbm.at[pl.ds(base * HEAD, CH * HEAD)],
                             eb, gsem),
        ]
        for j in range(NSUB):
            descs.append(pltpu.async_copy(
                v_hbm.at[cols_all.at[pl.ds(ci * CH + j * SG, SG)]],
                vb.at[pl.ds(j * SG, SG)], gsem))
        for d in descs:
            d.wait()

        def group(g, __):
            scale(vb, eb, g)
            return 0
        lax.fori_loop(0, NG, group, 0)

        for j in range(NG):
            rv = plsc.load_gather(rows_c, [lanes + j * 16])
            rsub[j // (SG // 16)][pl.ds((j % (SG // 16)) * 16, 16)] = rv

        odescs = []
        for j in range(NSUB):
            odescs.append(pltpu.async_copy(
                vb.at[pl.ds(j * SG, SG)], res_sh.at[rsub[j]], osem,
                add=True))
        for d in odescs:
            d.wait()
        return 0

    lax.fori_loop(0, NCHUNK, chunk, 0)

    tb = wbase + TB
    dts = [
        pltpu.async_copy(rows_hbm.at[pl.ds(tb, CT)], rows_t, gsem),
        pltpu.async_copy(exp_hbm.at[pl.ds(tb * HEAD, CT * HEAD)],
                         eb.at[pl.ds(0, CT * HEAD)], gsem),
        pltpu.async_copy(v_hbm.at[cols_all.at[pl.ds(TB, CT)]], vt, gsem),
    ]
    for d in dts:
        d.wait()
    scale(vt, eb, 0)
    pltpu.async_copy(vt, res_sh.at[rows_t], osem, add=True).wait()

    plsc.subcore_barrier()
    pltpu.sync_copy(res_sh.at[pl.ds(p0, RPT)],
                    resp_hbm.at[cid, pl.ds(p0, RPT)])

    @pl.when(sid == NS - 1)
    def _():
        pltpu.sync_copy(res_sh.at[pl.ds(t0, RTAIL)],
                        resp_hbm.at[cid, pl.ds(t0, RTAIL)])


def _agg_pass(v, rows, cols, exp_flat):
    mesh = plsc.VectorSubcoreMesh(core_axis_name="c", subcore_axis_name="s")
    zres = jnp.zeros((N, DIM), jnp.float32)
    fn = functools.partial(
        pl.kernel,
        mesh=mesh,
        compiler_params=_SC_PARAMS,
        out_type=jax.ShapeDtypeStruct((NC, N, DIM), jnp.float32),
        scratch_types=[
            pltpu.VMEM((EPW,), jnp.int32),
            pltpu.VMEM((CH,), jnp.int32),
            pltpu.VMEM((CT,), jnp.int32),
            pltpu.VMEM((CH, DIM), jnp.float32),
            pltpu.VMEM((CT, DIM), jnp.float32),
            pltpu.VMEM((CH * HEAD,), jnp.float32),
            pltpu.VMEM((SG,), jnp.int32),
            pltpu.VMEM((SG,), jnp.int32),
            pltpu.VMEM((SG,), jnp.int32),
            pltpu.VMEM((SG,), jnp.int32),
            pltpu.VMEM_SHARED((N, DIM), jnp.float32),
            pltpu.SemaphoreType.DMA,
            pltpu.SemaphoreType.DMA,
        ],
    )(_agg_body)
    return fn(v, rows, cols, exp_flat, zres)


def _comb_body(resp_ref, normp_ref, res_ref, rn_ref):
    nsum = jnp.sum(normp_ref[...], axis=0)
    rn = 1.0 / (nsum + 1e-8)
    rn_ref[...] = rn
    hsel = (lax.broadcasted_iota(jnp.int32, (HEAD, DIM), 0)
            == lax.broadcasted_iota(jnp.int32, (HEAD, DIM), 1) // DH)
    denom = jnp.dot(rn, hsel.astype(jnp.float32),
                    preferred_element_type=jnp.float32)
    res_ref[...] = (resp_ref[0] + resp_ref[1]) * denom


def _combine(resp, normp):
    blk = 1000
    return pl.pallas_call(
        _comb_body,
        grid=(N // blk,),
        in_specs=[
            pl.BlockSpec((NC, blk, DIM), lambda i: (0, i, 0)),
            pl.BlockSpec((NW, blk, HEAD), lambda i: (0, i, 0)),
        ],
        out_specs=[
            pl.BlockSpec((blk, DIM), lambda i: (i, 0)),
            pl.BlockSpec((blk, HEAD), lambda i: (i, 0)),
        ],
        out_shape=[
            jax.ShapeDtypeStruct((N, DIM), jnp.float32),
            jax.ShapeDtypeStruct((N, HEAD), jnp.float32),
        ],
    )(resp, normp)


CH4 = 2000
NCHUNK4 = EPW // CH4


def _att_body(exp_hbm, rows_hbm, rn_hbm, att_hbm, rn_v, rows_v, exp_v, out_v,
              gsem):
    cid = lax.axis_index("c")
    sid = lax.axis_index("s")
    wid = sid * NC + cid
    pltpu.sync_copy(rn_hbm, rn_v)
    lanes = lax.iota(jnp.int32, 16)

    def chunk(ci, _):
        base = wid * EPW + ci * CH4
        d0 = pltpu.async_copy(rows_hbm.at[pl.ds(base, CH4)], rows_v, gsem)
        d1 = pltpu.async_copy(exp_hbm.at[pl.ds(base * HEAD, CH4 * HEAD)],
                              exp_v, gsem)
        d0.wait()
        d1.wait()

        def group(g, __):
            eidx = lanes + g * 16
            ri = plsc.load_gather(rows_v, [eidx])
            for h in range(HEAD):
                ev = plsc.load_gather(exp_v, [g * 64 + h * 16 + lanes])
                rv = plsc.load_gather(rn_v, [ri * HEAD + h])
                plsc.store_scatter(out_v, [eidx * HEAD + h], ev * rv)
            return 0

        lax.fori_loop(0, CH4 // 16, group, 0)
        pltpu.async_copy(out_v,
                         att_hbm.at[pl.ds(base * HEAD, CH4 * HEAD)],
                         gsem).wait()
        return 0

    lax.fori_loop(0, NCHUNK4, chunk, 0)


def _att_norm(exp_flat, rows, rn_flat):
    mesh = plsc.VectorSubcoreMesh(core_axis_name="c", subcore_axis_name="s")
    fn = functools.partial(
        pl.kernel,
        mesh=mesh,
        compiler_params=_SC_PARAMS,
        out_type=jax.ShapeDtypeStruct((E * HEAD,), jnp.float32),
        scratch_types=[
            pltpu.VMEM((N4,), jnp.float32),
            pltpu.VMEM((CH4,), jnp.int32),
            pltpu.VMEM((CH4 * HEAD,), jnp.float32),
            pltpu.VMEM((CH4 * HEAD,), jnp.float32),
            pltpu.SemaphoreType.DMA,
        ],
    )(_att_body)
    return fn(exp_flat, rows, rn_flat)


def kernel(embeds, edge_index, qW, kW, vW):
    rows = edge_index[0]
    cols = edge_index[1]
    wall = jnp.concatenate([qW, kW, vW], axis=1)
    qkv = _qkv(embeds, wall)
    q = qkv[:, :DIM]
    k = qkv[:, DIM:2 * DIM]
    v = qkv[:, 2 * DIM:]
    exp_flat, normp = _score_pass(q, k, rows, cols)
    resp = _agg_pass(v, rows, cols, exp_flat)
    res, rn = _combine(resp, normp.reshape(NW, N, HEAD))
    att = _att_norm(exp_flat, rows, rn.reshape(N4))
    return res, att.reshape(E, HEAD)


# speedup vs baseline: 3.8937x; 1.0067x over previous
import functools

import jax
import jax.numpy as jnp
from jax import lax
from jax.experimental import pallas as pl
from jax.experimental.pallas import tpu as pltpu
from jax.experimental.pallas import tpu_sc as plsc

N = 10000
E = 320000
DIM = 128
HEAD = 4
DH = DIM // HEAD
N4 = N * HEAD

NC = 2
NS = 16
NW = NC * NS
EPW = E // NW
CH = 192
NG = CH // 16
SG = 48
NSUB = CH // SG
NCHUNK = EPW // CH
TB = NCHUNK * CH
CT = EPW - TB
RPT = 624
RTAIL = N - NS * RPT

_SC_PARAMS = pltpu.CompilerParams(needs_layout_passes=False)


def _qkv_body(e_ref, w_ref, q_ref, k_ref, v_ref):
    qkv = jnp.dot(e_ref[...], w_ref[...],
                  preferred_element_type=jnp.float32)
    q_ref[...] = qkv[:, :DIM]
    k_ref[...] = qkv[:, DIM:2 * DIM]
    v_ref[...] = qkv[:, 2 * DIM:]


def _qkv(embeds, wall):
    blk = 1000
    out = pl.BlockSpec((blk, DIM), lambda i: (i, 0))
    return pl.pallas_call(
        _qkv_body,
        grid=(N // blk,),
        in_specs=[
            pl.BlockSpec((blk, DIM), lambda i: (i, 0)),
            pl.BlockSpec((DIM, 3 * DIM), lambda i: (0, 0)),
        ],
        out_specs=[out, out, out],
        out_shape=[jax.ShapeDtypeStruct((N, DIM), jnp.float32)] * 3,
    )(embeds, wall)


def _score_body(q_hbm, k_hbm, rows_hbm, cols_hbm,
                exp_hbm, normp_hbm,
                rows_all, cols_all, qb, kb, qt, kt, eb, ntab, gsem, osem):
    cid = lax.axis_index("c")
    sid = lax.axis_index("s")
    wid = sid * NC + cid
    wbase = wid * EPW
    lanes = lax.iota(jnp.int32, 16)
    zero16 = jnp.zeros((16,), jnp.float32)

    def ztab(i, _):
        ntab[pl.ds(i * 16, 16)] = zero16
        return 0
    lax.fori_loop(0, N4 // 16, ztab, 0)

    d0 = pltpu.async_copy(rows_hbm.at[pl.ds(wbase, EPW)], rows_all, gsem)
    d1 = pltpu.async_copy(cols_hbm.at[pl.ds(wbase, EPW)], cols_all, gsem)
    d0.wait()
    d1.wait()

    def heads(buf_q, buf_k, loc0, g):
        eloc = lanes + g * 16
        eabs = eloc + loc0
        for h in range(HEAD):
            def dot_d(d2, acc):
                dd = d2 * 4
                for t in range(4):
                    d = (h * DH) + ((lanes + dd + t) & (DH - 1))
                    qc = plsc.load_gather(buf_q, [eloc, d])
                    kc = plsc.load_gather(buf_k, [eloc, d])
                    acc = acc + qc * kc
                return acc
            att = lax.fori_loop(0, DH // 4, dot_d,
                                jnp.zeros((16,), jnp.float32))
            att = jnp.exp(jnp.clip(att, -10.0, 10.0))
            eb[pl.ds(g * 64 + h * 16, 16)] = att
            rv = plsc.load_gather(rows_all, [eabs])
            plsc.addupdate_scatter(ntab, [rv * HEAD + h], att)

    def chunk(ci, _):
        loc = ci * CH
        descs = []
        for j in range(NSUB):
            descs.append(pltpu.async_copy(
                q_hbm.at[rows_all.at[pl.ds(loc + j * SG, SG)]],
                qb.at[pl.ds(j * SG, SG)], gsem))
            descs.append(pltpu.async_copy(
                k_hbm.at[cols_all.at[pl.ds(loc + j * SG, SG)]],
                kb.at[pl.ds(j * SG, SG)], gsem))
        for d in descs:
            d.wait()

        def group(g, __):
            heads(qb, kb, loc, g)
            return 0
        lax.fori_loop(0, NG, group, 0)

        od = pltpu.async_copy(
            eb, exp_hbm.at[pl.ds((wbase + ci * CH) * HEAD, CH * HEAD)], osem)
        od.wait()
        return 0

    lax.fori_loop(0, NCHUNK, chunk, 0)

    dt0 = pltpu.async_copy(q_hbm.at[rows_all.at[pl.ds(TB, CT)]], qt, gsem)
    dt1 = pltpu.async_copy(k_hbm.at[cols_all.at[pl.ds(TB, CT)]], kt, gsem)
    dt0.wait()
    dt1.wait()
    heads(qt, kt, TB, 0)
    pltpu.async_copy(eb.at[pl.ds(0, CT * HEAD)],
                     exp_hbm.at[pl.ds((wbase + TB) * HEAD, CT * HEAD)],
                     osem).wait()

    pltpu.sync_copy(ntab, normp_hbm.at[pl.ds(wid * N4, N4)])


def _score_pass(q, k, rows, cols):
    mesh = plsc.VectorSubcoreMesh(core_axis_name="c", subcore_axis_name="s")
    fn = functools.partial(
        pl.kernel,
        mesh=mesh,
        compiler_params=_SC_PARAMS,
        out_type=[
            jax.ShapeDtypeStruct((E * HEAD,), jnp.float32),
            jax.ShapeDtypeStruct((NW * N4,), jnp.float32),
        ],
        scratch_types=[
            pltpu.VMEM((EPW,), jnp.int32),
            pltpu.VMEM((EPW,), jnp.int32),
            pltpu.VMEM((CH, DIM), jnp.float32),
            pltpu.VMEM((CH, DIM), jnp.float32),
            pltpu.VMEM((CT, DIM), jnp.float32),
            pltpu.VMEM((CT, DIM), jnp.float32),
            pltpu.VMEM((CH * HEAD,), jnp.float32),
            pltpu.VMEM((N4,), jnp.float32),
            pltpu.SemaphoreType.DMA,
            pltpu.SemaphoreType.DMA,
        ],
    )(_score_body)
    return fn(q, k, rows, cols)


def _agg_body(v_hbm, rows_hbm, cols_hbm, exp_hbm, zres_hbm,
              resp_hbm,
              cols_all, rows_c, rows_t, vb, vt, eb,
              rs0, rs1, rs2, rs3,
              res_sh, gsem, osem):
    cid = lax.axis_index("c")
    sid = lax.axis_index("s")
    wid = sid * NC + cid
    wbase = wid * EPW
    lanes = lax.iota(jnp.int32, 16)
    rsub = [rs0, rs1, rs2, rs3]

    t0 = NS * RPT
    p0 = sid * RPT
    pltpu.sync_copy(zres_hbm.at[pl.ds(p0, RPT)], res_sh.at[pl.ds(p0, RPT)])

    @pl.when(sid == NS - 1)
    def _():
        pltpu.sync_copy(zres_hbm.at[pl.ds(t0, RTAIL)],
                        res_sh.at[pl.ds(t0, RTAIL)])

    d0 = pltpu.async_copy(cols_hbm.at[pl.ds(wbase, EPW)], cols_all, gsem)
    d0.wait()
    plsc.subcore_barrier()

    def scale(buf_v, buf_e, g):
        eidx = lanes + g * 16
        for h in range(HEAD):
            att = plsc.load_gather(buf_e, [g * 64 + h * 16 + lanes])

            def scale_d(d2, tt):
                dd = d2 * 4
                for t in range(4):
                    d = (h * DH) + ((lanes + dd + t) & (DH - 1))
                    vc = plsc.load_gather(buf_v, [eidx, d])
                    plsc.store_scatter(buf_v, [eidx, d], vc * att)
                return tt
            lax.fori_loop(0, DH // 4, scale_d, 0)

    def chunk(ci, _):
        base = wbase + ci * CH
        descs = [
            pltpu.async_copy(rows_hbm.at[pl.ds(base, CH)], rows_c, gsem),
            pltpu.async_copy(exp_hbm.at[pl.ds(base * HEAD, CH * HEAD)],
                             eb, gsem),
        ]
        for j in range(NSUB):
            descs.append(pltpu.async_copy(
                v_hbm.at[cols_all.at[pl.ds(ci * CH + j * SG, SG)]],
                vb.at[pl.ds(j * SG, SG)], gsem))
        for d in descs:
            d.wait()

        def group(g, __):
            scale(vb, eb, g)
            return 0
        lax.fori_loop(0, NG, group, 0)

        for j in range(NG):
            rv = plsc.load_gather(rows_c, [lanes + j * 16])
            rsub[j // (SG // 16)][pl.ds((j % (SG // 16)) * 16, 16)] = rv

        odescs = []
        for j in range(NSUB):
            odescs.append(pltpu.async_copy(
                vb.at[pl.ds(j * SG, SG)], res_sh.at[rsub[j]], osem,
                add=True))
        for d in odescs:
            d.wait()
        return 0

    lax.fori_loop(0, NCHUNK, chunk, 0)

    tb = wbase + TB
    dts = [
        pltpu.async_copy(rows_hbm.at[pl.ds(tb, CT)], rows_t, gsem),
        pltpu.async_copy(exp_hbm.at[pl.ds(tb * HEAD, CT * HEAD)],
                         eb.at[pl.ds(0, CT * HEAD)], gsem),
        pltpu.async_copy(v_hbm.at[cols_all.at[pl.ds(TB, CT)]], vt, gsem),
    ]
    for d in dts:
        d.wait()
    scale(vt, eb, 0)
    pltpu.async_copy(vt, res_sh.at[rows_t], osem, add=True).wait()

    plsc.subcore_barrier()
    pltpu.sync_copy(res_sh.at[pl.ds(p0, RPT)],
                    resp_hbm.at[cid, pl.ds(p0, RPT)])

    @pl.when(sid == NS - 1)
    def _():
        pltpu.sync_copy(res_sh.at[pl.ds(t0, RTAIL)],
                        resp_hbm.at[cid, pl.ds(t0, RTAIL)])


def _agg_pass(v, rows, cols, exp_flat):
    mesh = plsc.VectorSubcoreMesh(core_axis_name="c", subcore_axis_name="s")
    zres = jnp.zeros((N, DIM), jnp.float32)
    fn = functools.partial(
        pl.kernel,
        mesh=mesh,
        compiler_params=_SC_PARAMS,
        out_type=jax.ShapeDtypeStruct((NC, N, DIM), jnp.float32),
        scratch_types=[
            pltpu.VMEM((EPW,), jnp.int32),
            pltpu.VMEM((CH,), jnp.int32),
            pltpu.VMEM((CT,), jnp.int32),
            pltpu.VMEM((CH, DIM), jnp.float32),
            pltpu.VMEM((CT, DIM), jnp.float32),
            pltpu.VMEM((CH * HEAD,), jnp.float32),
            pltpu.VMEM((SG,), jnp.int32),
            pltpu.VMEM((SG,), jnp.int32),
            pltpu.VMEM((SG,), jnp.int32),
            pltpu.VMEM((SG,), jnp.int32),
            pltpu.VMEM_SHARED((N, DIM), jnp.float32),
            pltpu.SemaphoreType.DMA,
            pltpu.SemaphoreType.DMA,
        ],
    )(_agg_body)
    return fn(v, rows, cols, exp_flat, zres)


def _comb_body(resp_ref, normp_ref, res_ref, rn_ref):
    nsum = jnp.sum(normp_ref[...], axis=0)
    rn = 1.0 / (nsum + 1e-8)
    rn_ref[...] = rn
    hsel = (lax.broadcasted_iota(jnp.int32, (HEAD, DIM), 0)
            == lax.broadcasted_iota(jnp.int32, (HEAD, DIM), 1) // DH)
    denom = jnp.dot(rn, hsel.astype(jnp.float32),
                    preferred_element_type=jnp.float32)
    res_ref[...] = (resp_ref[0] + resp_ref[1]) * denom


def _combine(resp, normp):
    blk = 1000
    return pl.pallas_call(
        _comb_body,
        grid=(N // blk,),
        in_specs=[
            pl.BlockSpec((NC, blk, DIM), lambda i: (0, i, 0)),
            pl.BlockSpec((NW, blk, HEAD), lambda i: (0, i, 0)),
        ],
        out_specs=[
            pl.BlockSpec((blk, DIM), lambda i: (i, 0)),
            pl.BlockSpec((blk, HEAD), lambda i: (i, 0)),
        ],
        out_shape=[
            jax.ShapeDtypeStruct((N, DIM), jnp.float32),
            jax.ShapeDtypeStruct((N, HEAD), jnp.float32),
        ],
    )(resp, normp)


CH4 = 2000
NCHUNK4 = EPW // CH4


def _att_body(exp_hbm, rows_hbm, rn_hbm, att_hbm, rn_v, rows_v, exp_v, out_v,
              gsem):
    cid = lax.axis_index("c")
    sid = lax.axis_index("s")
    wid = sid * NC + cid
    pltpu.sync_copy(rn_hbm, rn_v)
    lanes = lax.iota(jnp.int32, 16)

    def chunk(ci, _):
        base = wid * EPW + ci * CH4
        d0 = pltpu.async_copy(rows_hbm.at[pl.ds(base, CH4)], rows_v, gsem)
        d1 = pltpu.async_copy(exp_hbm.at[pl.ds(base * HEAD, CH4 * HEAD)],
                              exp_v, gsem)
        d0.wait()
        d1.wait()

        def group(g, __):
            eidx = lanes + g * 16
            ri = plsc.load_gather(rows_v, [eidx])
            for h in range(HEAD):
                ev = plsc.load_gather(exp_v, [g * 64 + h * 16 + lanes])
                rv = plsc.load_gather(rn_v, [ri * HEAD + h])
                plsc.store_scatter(out_v, [eidx * HEAD + h], ev * rv)
            return 0

        lax.fori_loop(0, CH4 // 16, group, 0)
        pltpu.async_copy(out_v,
                         att_hbm.at[pl.ds(base * HEAD, CH4 * HEAD)],
                         gsem).wait()
        return 0

    lax.fori_loop(0, NCHUNK4, chunk, 0)


def _att_norm(exp_flat, rows, rn_flat):
    mesh = plsc.VectorSubcoreMesh(core_axis_name="c", subcore_axis_name="s")
    fn = functools.partial(
        pl.kernel,
        mesh=mesh,
        compiler_params=_SC_PARAMS,
        out_type=jax.ShapeDtypeStruct((E * HEAD,), jnp.float32),
        scratch_types=[
            pltpu.VMEM((N4,), jnp.float32),
            pltpu.VMEM((CH4,), jnp.int32),
            pltpu.VMEM((CH4 * HEAD,), jnp.float32),
            pltpu.VMEM((CH4 * HEAD,), jnp.float32),
            pltpu.SemaphoreType.DMA,
        ],
    )(_att_body)
    return fn(exp_flat, rows, rn_flat)


def kernel(embeds, edge_index, qW, kW, vW):
    rows = edge_index[0]
    cols = edge_index[1]
    wall = jnp.concatenate([qW, kW, vW], axis=1)
    q, k, v = _qkv(embeds, wall)
    exp_flat, normp = _score_pass(q, k, rows, cols)
    resp = _agg_pass(v, rows, cols, exp_flat)
    res, rn = _combine(resp, normp.reshape(NW, N, HEAD))
    att = _att_norm(exp_flat, rows, rn.reshape(N4))
    return res, att.reshape(E, HEAD)
